# baseline (speedup 1.0000x reference)
import dataclasses
import functools

import jax
import jax.numpy as jnp
from jax import lax
from jax.experimental import pallas as pl
from jax.experimental.pallas import tpu as pltpu
from jax.experimental.pallas import tpu_sc as plsc

B, N, H, K = 4, 2048, 256, 16
BN = B * N
ALPHA, BETA, GAMMA_CROSS = 0.6, 0.4, 1.2
INF = 1e9
R = 256
NB = N // R
C = 8
LANES = 16


def _rne_bf16(x):
    y = lax.bitcast_convert_type(x, jnp.uint32)
    r = (y + 0x7FFF + ((y >> 16) & 1)) & jnp.uint32(0xFFFF0000)
    return lax.bitcast_convert_type(r, jnp.float32)


def _build_graph_kernel(crow_ref, ctall_ref, hrow_ref, hall_ref, trow_ref,
                        tall_ref, idx_ref, w_ref):
    b = pl.program_id(0)
    r = pl.program_id(1)
    crow = crow_ref[0]
    ctall = ctall_ref[0]
    x_row = crow[:, 0:1]
    y_row = crow[:, 1:2]
    x_all = ctall[0:1, :]
    y_all = ctall[1:2, :]

    sq_row = x_row * x_row + y_row * y_row
    sq_all = x_all * x_all + y_all * y_all
    cross = (_rne_bf16(x_row) * _rne_bf16(x_all)
             + _rne_bf16(y_row) * _rne_bf16(y_all))
    d2 = sq_row + sq_all - 2.0 * cross
    dist = jnp.sqrt(jnp.maximum(d2, 0.0))

    col_ids = lax.broadcasted_iota(jnp.int32, (R, N), 1)
    row_ids = lax.broadcasted_iota(jnp.int32, (R, N), 0) + r * R
    dist = jnp.where(col_ids == row_ids, INF, dist)

    dd = dist
    mask = jnp.zeros((R, N), dtype=jnp.bool_)
    firsts = []
    for _ in range(K):
        m = jnp.min(dd, axis=1, keepdims=True)
        cand = dd == m
        first = jnp.min(jnp.where(cand, col_ids, N), axis=1, keepdims=True)
        sel = col_ids == first
        mask = mask | sel
        dd = jnp.where(sel, INF, dd)
        firsts.append(first)
    maskf = mask.astype(jnp.float32)

    inv_d = maskf / jnp.clip(dist, 1e-4, None)
    w_spatial = inv_d / jnp.clip(jnp.sum(inv_d, axis=1, keepdims=True),
                                 1e-8, None)

    hrow = hrow_ref[0]
    hall = hall_ref[0]
    hn_row = hrow * lax.rsqrt(
        jnp.clip(jnp.sum(hrow * hrow, axis=1, keepdims=True), 1e-24, None))
    hn_all = hall * lax.rsqrt(
        jnp.clip(jnp.sum(hall * hall, axis=1, keepdims=True), 1e-24, None))
    sim = lax.dot_general(hn_row, hn_all, (((1,), (1,)), ((), ())),
                          precision=lax.Precision.HIGHEST,
                          preferred_element_type=jnp.float32)
    sim = jnp.maximum(sim, 0.0) * maskf
    w_sem = sim / jnp.clip(jnp.sum(sim, axis=1, keepdims=True), 1e-8, None)

    t_row = trow_ref[0]
    t_all = tall_ref[0]
    is_cross = t_row != t_all
    a = (ALPHA * w_spatial + BETA * w_sem) * jnp.where(
        is_cross & mask, GAMMA_CROSS, 1.0)
    a = a / jnp.clip(jnp.sum(a, axis=1, keepdims=True), 1e-8, None)

    idx_cols, w_cols = [], []
    for kk in range(K):
        sel = col_ids == firsts[kk]
        w_k = jnp.sum(jnp.where(sel, a, 0.0), axis=1, keepdims=True)
        idx_cols.append(firsts[kk] + b * N)
        w_cols.append(w_k)
    idx_ref[0] = jnp.concatenate(idx_cols, axis=1)
    w_ref[0] = jnp.concatenate(w_cols, axis=1)


def _build_graph(centroids, h, time_ids):
    ct = centroids.transpose(0, 2, 1)
    t_col = time_ids.reshape(B, N, 1)
    t_row = time_ids.reshape(B, 1, N)
    return pl.pallas_call(
        _build_graph_kernel,
        grid=(B, NB),
        in_specs=[
            pl.BlockSpec((1, R, 2), lambda b, r: (b, r, 0)),
            pl.BlockSpec((1, 2, N), lambda b, r: (b, 0, 0)),
            pl.BlockSpec((1, R, H), lambda b, r: (b, r, 0)),
            pl.BlockSpec((1, N, H), lambda b, r: (b, 0, 0)),
            pl.BlockSpec((1, R, 1), lambda b, r: (b, r, 0)),
            pl.BlockSpec((1, 1, N), lambda b, r: (b, 0, 0)),
        ],
        out_specs=[
            pl.BlockSpec((1, R, K), lambda b, r: (b, r, 0)),
            pl.BlockSpec((1, R, K), lambda b, r: (b, r, 0)),
        ],
        out_shape=[
            jax.ShapeDtypeStruct((B, N, K), jnp.int32),
            jax.ShapeDtypeStruct((B, N, K), jnp.float32),
        ],
    )(centroids, ct, h, h, t_col, t_row)


def _sc_agg(h2d, gidx_row, w2d):
    mesh = plsc.VectorSubcoreMesh(core_axis_name="c", subcore_axis_name="s")
    cp = pltpu.CompilerParams()
    if "needs_layout_passes" in pltpu.CompilerParams.__dataclass_fields__:
        cp = dataclasses.replace(cp, needs_layout_passes=False)

    @functools.partial(
        pl.kernel,
        out_type=jax.ShapeDtypeStruct((BN, H), jnp.float32),
        mesh=mesh,
        scratch_types=[pltpu.VMEM((C * K, H), jnp.float32)],
        compiler_params=cp,
    )
    def k(h_hbm, i_hbm, w_hbm, o_hbm, rows_vmem):
        def body(i_vmem, w_vmem, o_vmem):
            pltpu.sync_copy(h_hbm.at[i_vmem.at[0]], rows_vmem)
            kconsts = [lax.iota(jnp.int32, LANES) * 0 + kk
                       for kk in range(K)]
            zero = jnp.zeros((LANES,), jnp.float32)

            @pl.loop(0, C)
            def _(n):
                idxn = lax.iota(jnp.int32, LANES) * 0 + n
                wks = [plsc.load_gather(w_vmem, [idxn, kconsts[kk]])
                       for kk in range(K)]
                for j in range(H // LANES):
                    acc = zero
                    for kk in range(K):
                        acc = acc + (rows_vmem[n * K + kk,
                                               pl.ds(j * LANES, LANES)]
                                     * wks[kk])
                    o_vmem[n, pl.ds(j * LANES, LANES)] = acc

        pltpu.emit_pipeline(
            body,
            grid=(BN // C,),
            in_specs=[
                pl.BlockSpec((1, C * K), lambda i: (0, i)),
                pl.BlockSpec((C, K), lambda i: (i, 0)),
            ],
            out_specs=[pl.BlockSpec((C, H), lambda i: (i, 0))],
            core_axis_name=("c", "s"),
            dimension_semantics=(pltpu.PARALLEL,),
        )(i_hbm, w_hbm, o_hbm)

    return k(h2d, gidx_row, w2d)


def _dense_kernel(hrow_ref, aggrow_ref, ws_ref, wn_ref, g_ref, b_ref, o_ref):
    hrow = hrow_ref[...]
    h_agg = aggrow_ref[...]
    z = (lax.dot_general(hrow, ws_ref[...], (((1,), (1,)), ((), ())),
                         precision=lax.Precision.HIGHEST,
                         preferred_element_type=jnp.float32)
         + lax.dot_general(h_agg, wn_ref[...], (((1,), (1,)), ((), ())),
                           precision=lax.Precision.HIGHEST,
                           preferred_element_type=jnp.float32))
    out = 0.5 * z * (1.0 + lax.erf(z * 0.7071067811865476))
    mu = jnp.mean(out, axis=1, keepdims=True)
    xc = out - mu
    var = jnp.mean(xc * xc, axis=1, keepdims=True)
    y = xc * lax.rsqrt(var + 1e-5) * g_ref[...] + b_ref[...]
    o_ref[...] = hrow + y


def _dense_layer(h2d, agg2d, w_self, w_neigh, g, b):
    return pl.pallas_call(
        _dense_kernel,
        grid=(BN // R,),
        in_specs=[
            pl.BlockSpec((R, H), lambda i: (i, 0)),
            pl.BlockSpec((R, H), lambda i: (i, 0)),
            pl.BlockSpec((H, H), lambda i: (0, 0)),
            pl.BlockSpec((H, H), lambda i: (0, 0)),
            pl.BlockSpec((1, H), lambda i: (0, 0)),
            pl.BlockSpec((1, H), lambda i: (0, 0)),
        ],
        out_specs=pl.BlockSpec((R, H), lambda i: (i, 0)),
        out_shape=jax.ShapeDtypeStruct((BN, H), jnp.float32),
    )(h2d, agg2d, w_self, w_neigh, g.reshape(1, H), b.reshape(1, H))


def kernel(repr_pad, padding_mask, centroids_pad, time_ids_pad, W_self0,
           W_neigh0, ln_g0, ln_b0, W_self1, W_neigh1, ln_g1, ln_b1):
    gidx, w = _build_graph(centroids_pad.astype(jnp.float32),
                           repr_pad.astype(jnp.float32), time_ids_pad)
    gidx_row = gidx.reshape(1, BN * K)
    w2d = w.reshape(BN, K)
    h2d = repr_pad.reshape(BN, H)
    agg0 = _sc_agg(h2d, gidx_row, w2d)
    out0 = _dense_layer(h2d, agg0, W_self0, W_neigh0, ln_g0, ln_b0)
    agg1 = _sc_agg(out0, gidx_row, w2d)
    out1 = _dense_layer(out0, agg1, W_self1, W_neigh1, ln_g1, ln_b1)
    out = out1.reshape(B, N, H)
    return jnp.where(padding_mask[..., None], 0.0, out)


# speedup vs baseline: 2.2129x; 2.2129x over previous
import jax
import jax.numpy as jnp
from jax import lax
from jax.experimental import pallas as pl

B, N, H, K = 4, 2048, 256, 16
ALPHA, BETA, GAMMA_CROSS = 0.6, 0.4, 1.2
INF = 1e9
R = 256
NB = N // R


def _rne_bf16(x):
    y = lax.bitcast_convert_type(x, jnp.uint32)
    r = (y + 0x7FFF + ((y >> 16) & 1)) & jnp.uint32(0xFFFF0000)
    return lax.bitcast_convert_type(r, jnp.float32)


def _dense_tail(hrow, h_agg, ws, wn, g, b):
    z = (lax.dot_general(hrow, ws, (((1,), (1,)), ((), ())),
                         preferred_element_type=jnp.float32)
         + lax.dot_general(h_agg, wn, (((1,), (1,)), ((), ())),
                           preferred_element_type=jnp.float32))
    out = 0.5 * z * (1.0 + lax.erf(z * 0.7071067811865476))
    mu = jnp.mean(out, axis=1, keepdims=True)
    xc = out - mu
    var = jnp.mean(xc * xc, axis=1, keepdims=True)
    return hrow + xc * lax.rsqrt(var + 1e-5) * g + b


def _build_kernel(crow_ref, ctall_ref, hrow_ref, hall_ref, trow_ref,
                  tall_ref, ws_ref, wn_ref, g_ref, b_ref, a_ref, o_ref):
    r = pl.program_id(1)
    crow = crow_ref[0]
    ctall = ctall_ref[0]
    x_row = crow[:, 0:1]
    y_row = crow[:, 1:2]
    x_all = ctall[0:1, :]
    y_all = ctall[1:2, :]

    sq_row = x_row * x_row + y_row * y_row
    sq_all = x_all * x_all + y_all * y_all
    cross = (_rne_bf16(x_row) * _rne_bf16(x_all)
             + _rne_bf16(y_row) * _rne_bf16(y_all))
    d2 = sq_row + sq_all - 2.0 * cross
    dist = jnp.sqrt(jnp.maximum(d2, 0.0))

    col_ids = lax.broadcasted_iota(jnp.int32, (R, N), 1)
    row_ids = lax.broadcasted_iota(jnp.int32, (R, N), 0) + r * R
    dist = jnp.where(col_ids == row_ids, INF, dist)

    colf1 = (col_ids + 1).astype(jnp.float32)
    distk = jnp.where(dist == 0.0, 1e-24 * colf1, dist)
    dd = distk
    m = None
    for _ in range(K):
        m = jnp.min(dd, axis=1, keepdims=True)
        dd = jnp.where(dd == m, INF, dd)
    mask = distk <= m
    maskf = mask.astype(jnp.float32)

    inv_d = maskf / jnp.clip(dist, 1e-4, None)
    w_spatial = inv_d / jnp.clip(jnp.sum(inv_d, axis=1, keepdims=True),
                                 1e-8, None)

    hrow = hrow_ref[0]
    hall = hall_ref[0]
    hn_row = hrow * lax.rsqrt(
        jnp.clip(jnp.sum(hrow * hrow, axis=1, keepdims=True), 1e-24, None))
    hn_all = hall * lax.rsqrt(
        jnp.clip(jnp.sum(hall * hall, axis=1, keepdims=True), 1e-24, None))
    sim = lax.dot_general(hn_row, hn_all, (((1,), (1,)), ((), ())),
                          precision=lax.Precision.HIGHEST,
                          preferred_element_type=jnp.float32)
    sim = jnp.maximum(sim, 0.0) * maskf
    w_sem = sim / jnp.clip(jnp.sum(sim, axis=1, keepdims=True), 1e-8, None)

    t_row = trow_ref[0]
    t_all = tall_ref[0]
    is_cross = t_row != t_all
    a = (ALPHA * w_spatial + BETA * w_sem) * jnp.where(
        is_cross & mask, GAMMA_CROSS, 1.0)
    a = a / jnp.clip(jnp.sum(a, axis=1, keepdims=True), 1e-8, None)
    a_ref[0] = a

    h_agg = lax.dot_general(a, hall, (((1,), (0,)), ((), ())),
                            precision=lax.Precision.HIGHEST,
                            preferred_element_type=jnp.float32)
    o_ref[0] = _dense_tail(hrow, h_agg, ws_ref[...], wn_ref[...],
                           g_ref[...], b_ref[...])


def _sage_kernel(a_ref, hrow_ref, hall_ref, ws_ref, wn_ref, g_ref, b_ref,
                 o_ref):
    a = a_ref[0]
    hrow = hrow_ref[0]
    hall = hall_ref[0]
    h_agg = lax.dot_general(a, hall, (((1,), (0,)), ((), ())),
                            precision=lax.Precision.HIGHEST,
                            preferred_element_type=jnp.float32)
    o_ref[0] = _dense_tail(hrow, h_agg, ws_ref[...], wn_ref[...],
                           g_ref[...], b_ref[...])


def kernel(repr_pad, padding_mask, centroids_pad, time_ids_pad, W_self0,
           W_neigh0, ln_g0, ln_b0, W_self1, W_neigh1, ln_g1, ln_b1):
    c = centroids_pad.astype(jnp.float32)
    h = repr_pad.astype(jnp.float32)
    ct = c.transpose(0, 2, 1)
    t_col = time_ids_pad.reshape(B, N, 1)
    t_row = time_ids_pad.reshape(B, 1, N)
    adj, out0 = pl.pallas_call(
        _build_kernel,
        grid=(B, NB),
        in_specs=[
            pl.BlockSpec((1, R, 2), lambda b, r: (b, r, 0)),
            pl.BlockSpec((1, 2, N), lambda b, r: (b, 0, 0)),
            pl.BlockSpec((1, R, H), lambda b, r: (b, r, 0)),
            pl.BlockSpec((1, N, H), lambda b, r: (b, 0, 0)),
            pl.BlockSpec((1, R, 1), lambda b, r: (b, r, 0)),
            pl.BlockSpec((1, 1, N), lambda b, r: (b, 0, 0)),
            pl.BlockSpec((H, H), lambda b, r: (0, 0)),
            pl.BlockSpec((H, H), lambda b, r: (0, 0)),
            pl.BlockSpec((1, H), lambda b, r: (0, 0)),
            pl.BlockSpec((1, H), lambda b, r: (0, 0)),
        ],
        out_specs=[
            pl.BlockSpec((1, R, N), lambda b, r: (b, r, 0)),
            pl.BlockSpec((1, R, H), lambda b, r: (b, r, 0)),
        ],
        out_shape=[
            jax.ShapeDtypeStruct((B, N, N), jnp.float32),
            jax.ShapeDtypeStruct((B, N, H), jnp.float32),
        ],
    )(c, ct, h, h, t_col, t_row, W_self0, W_neigh0,
      ln_g0.reshape(1, H), ln_b0.reshape(1, H))

    out = pl.pallas_call(
        _sage_kernel,
        grid=(B, NB),
        in_specs=[
            pl.BlockSpec((1, R, N), lambda b, r: (b, r, 0)),
            pl.BlockSpec((1, R, H), lambda b, r: (b, r, 0)),
            pl.BlockSpec((1, N, H), lambda b, r: (b, 0, 0)),
            pl.BlockSpec((H, H), lambda b, r: (0, 0)),
            pl.BlockSpec((H, H), lambda b, r: (0, 0)),
            pl.BlockSpec((1, H), lambda b, r: (0, 0)),
            pl.BlockSpec((1, H), lambda b, r: (0, 0)),
        ],
        out_specs=pl.BlockSpec((1, R, H), lambda b, r: (b, r, 0)),
        out_shape=jax.ShapeDtypeStruct((B, N, H), jnp.float32),
    )(adj, out0, out0, W_self1, W_neigh1,
      ln_g1.reshape(1, H), ln_b1.reshape(1, H))
    return jnp.where(padding_mask[..., None], 0.0, out)
